# baseline (speedup 1.0000x reference)
import functools
import math

import jax
import jax.numpy as jnp
from jax import lax
from jax.experimental import pallas as pl
from jax.experimental.pallas import tpu as pltpu
from jax.experimental.pallas import tpu_sc as plsc

B, T, D = 2, 4096, 1024
MAX_NGRAM = 2
NUM_HEADS = 4
TABLE = 100000
D_MEM = 512
HEADS_TOTAL = MAX_NGRAM * NUM_HEADS
D_HEAD = D_MEM // HEADS_TOTAL
KSZ = 4
DIL = MAX_NGRAM
EPS = 1e-5

N_TOK = B * T
NW = 32
TOK_PER_W = N_TOK // NW
CHUNK = 128
ROWS_PER_CHUNK = CHUNK * HEADS_TOTAL
N_CHUNKS = TOK_PER_W // CHUNK

TB = 512
HIST = (KSZ - 1) * DIL


def _sc_hash_gather(ids_u32, prev_u32, emb_table):
    mesh = plsc.VectorSubcoreMesh(core_axis_name="c", subcore_axis_name="s")

    @functools.partial(
        pl.kernel,
        out_type=jax.ShapeDtypeStruct((HEADS_TOTAL, N_TOK, D_HEAD), jnp.float32),
        mesh=mesh,
        scratch_types=[
            pltpu.VMEM((CHUNK,), jnp.uint32),
            pltpu.VMEM((CHUNK,), jnp.uint32),
            pltpu.VMEM((ROWS_PER_CHUNK + 16,), jnp.int32),
            pltpu.SemaphoreType.DMA,
        ],
    )
    def sc_kernel(ids_hbm, prev_hbm, table_hbm, out_hbm,
                  ids_v, prev_v, idx_v, sem):
        wid = lax.axis_index("s") * 2 + lax.axis_index("c")
        for c in range(N_CHUNKS):
            tb = wid * TOK_PER_W + c * CHUNK
            pltpu.sync_copy(ids_hbm.at[pl.ds(tb, CHUNK)], ids_v)
            pltpu.sync_copy(prev_hbm.at[pl.ds(tb, CHUNK)], prev_v)
            for g in range(CHUNK // 16):
                cur = ids_v[pl.ds(g * 16, 16)]
                prv = prev_v[pl.ds(g * 16, 16)]
                acc2 = cur * jnp.uint32(1000003) + prv
                for head in range(HEADS_TOTAL):
                    n, h = divmod(head, NUM_HEADS)
                    mult = jnp.uint32((2654435761 + 40503 * (h + 1)) & 0xFFFFFFFF)
                    acc = acc2 if n else cur
                    hv = (acc * mult) >> jnp.uint32(8)
                    fid = (hv % jnp.uint32(TABLE)).astype(jnp.int32) + head * TABLE
                    idx_v[pl.ds(head * CHUNK + g * 16, 16)] = fid
            for head in range(HEADS_TOTAL):
                out_h = out_hbm.at[head]

                def dma_body(t, _, head=head, out_h=out_h, tb=tb):
                    row = idx_v[pl.ds(head * CHUNK + t, 16)][0]
                    pltpu.async_copy(
                        table_hbm.at[pl.ds(row, 1)],
                        out_h.at[pl.ds(tb + t, 1)],
                        sem,
                    )
                    return _

                lax.fori_loop(0, CHUNK, dma_body, 0)
            pltpu.make_async_copy(
                table_hbm.at[pl.ds(0, ROWS_PER_CHUNK)],
                out_hbm.at[0].at[pl.ds(0, ROWS_PER_CHUNK)],
                sem,
            ).wait()

    return sc_kernel(ids_u32, prev_u32, emb_table)


def _tc_body(e_ref, h_ref, wkT_ref, bk_ref, wvT_ref, bv_ref, qw_ref, kw_ref,
             cnw_ref, conv4_ref, o_ref, hist_ref):
    t = pl.program_id(1)

    @pl.when(t == 0)
    def _():
        hist_ref[...] = jnp.zeros_like(hist_ref)

    k = bk_ref[...]
    v = bv_ref[...]
    for head in range(HEADS_TOTAL):
        e_h = e_ref[head, 0]
        k = k + jnp.dot(e_h, wkT_ref[head], preferred_element_type=jnp.float32)
        v = v + jnp.dot(e_h, wvT_ref[head], preferred_element_type=jnp.float32)
    h = h_ref[0]
    qn = h * lax.rsqrt(jnp.mean(h * h, -1, keepdims=True) + EPS) * qw_ref[...]
    kn = k * lax.rsqrt(jnp.mean(k * k, -1, keepdims=True) + EPS) * kw_ref[...]
    dot = jnp.sum(qn * kn, -1, keepdims=True) * (1.0 / math.sqrt(D))
    s = jnp.sqrt(jnp.maximum(jnp.abs(dot), 1e-6)) * jnp.sign(dot)
    gate = jax.nn.sigmoid(s)
    gv = gate * v
    gn = gv * lax.rsqrt(jnp.mean(gv * gv, -1, keepdims=True) + EPS) * cnw_ref[...]
    full = jnp.concatenate([hist_ref[8 - HIST:8], gn], axis=0)
    yc = (conv4_ref[0:1] * full[0:TB]
          + conv4_ref[1:2] * full[2:TB + 2]
          + conv4_ref[2:3] * full[4:TB + 4]
          + conv4_ref[3:4] * full[6:TB + 6])
    o_ref[0] = yc * jax.nn.sigmoid(yc) + gv
    hist_ref[8 - HIST:8] = gn[TB - HIST:TB]


def _tc_dense(e_hm, hidden, wkT3, bk2, wvT3, bv2, qw2, kw2, cnw2, conv4):
    grid = (B, T // TB)
    return pl.pallas_call(
        _tc_body,
        grid=grid,
        in_specs=[
            pl.BlockSpec((HEADS_TOTAL, 1, TB, D_HEAD), lambda b, t: (0, b, t, 0)),
            pl.BlockSpec((1, TB, D), lambda b, t: (b, t, 0)),
            pl.BlockSpec((HEADS_TOTAL, D_HEAD, D), lambda b, t: (0, 0, 0)),
            pl.BlockSpec((1, D), lambda b, t: (0, 0)),
            pl.BlockSpec((HEADS_TOTAL, D_HEAD, D), lambda b, t: (0, 0, 0)),
            pl.BlockSpec((1, D), lambda b, t: (0, 0)),
            pl.BlockSpec((1, D), lambda b, t: (0, 0)),
            pl.BlockSpec((1, D), lambda b, t: (0, 0)),
            pl.BlockSpec((1, D), lambda b, t: (0, 0)),
            pl.BlockSpec((KSZ, D), lambda b, t: (0, 0)),
        ],
        out_specs=pl.BlockSpec((1, TB, D), lambda b, t: (b, t, 0)),
        out_shape=jax.ShapeDtypeStruct((B, T, D), jnp.float32),
        scratch_shapes=[pltpu.VMEM((8, D), jnp.float32)],
    )(e_hm, hidden, wkT3, bk2, wvT3, bv2, qw2, kw2, cnw2, conv4)


def kernel(hidden_states, input_ids, emb_table, Wk, bk, Wv, bv, qn_w, kn_w, cn_w, conv_w):
    ids_u32 = input_ids.astype(jnp.uint32).reshape(-1)
    prev_u32 = jnp.concatenate(
        [jnp.zeros((B, 1), jnp.uint32), input_ids[:, :-1].astype(jnp.uint32)], axis=1
    ).reshape(-1)
    e_hm = _sc_hash_gather(ids_u32, prev_u32, emb_table)
    e_hm4 = e_hm.reshape(HEADS_TOTAL, B, T, D_HEAD)
    return _tc_dense(
        e_hm4,
        hidden_states,
        Wk.T.reshape(HEADS_TOTAL, D_HEAD, D),
        bk.reshape(1, D),
        Wv.T.reshape(HEADS_TOTAL, D_HEAD, D),
        bv.reshape(1, D),
        qn_w.reshape(1, D),
        kn_w.reshape(1, D),
        cn_w.reshape(1, D),
        jnp.transpose(conv_w[:, 0, :]),
    )


# speedup vs baseline: 2.2334x; 2.2334x over previous
import functools
import math

import jax
import jax.numpy as jnp
from jax import lax
from jax.experimental import pallas as pl
from jax.experimental.pallas import tpu as pltpu
from jax.experimental.pallas import tpu_sc as plsc

B, T, D = 2, 4096, 1024
MAX_NGRAM = 2
NUM_HEADS = 4
TABLE = 100000
D_MEM = 512
HEADS_TOTAL = MAX_NGRAM * NUM_HEADS
D_HEAD = D_MEM // HEADS_TOTAL
KSZ = 4
DIL = MAX_NGRAM
EPS = 1e-5

N_TOK = B * T
NW = 32
TOK_PER_W = N_TOK // NW
CHUNK = 128
ROWS_PER_CHUNK = CHUNK * HEADS_TOTAL
N_CHUNKS = TOK_PER_W // CHUNK

TB = 512
NT = T // TB
HIST = (KSZ - 1) * DIL
LB = TB // 2
N_LINES = N_TOK * HEADS_TOTAL * D_HEAD // 128


def _sc_hash_gather(ids_u32, prev_u32, emb_table):
    mesh = plsc.VectorSubcoreMesh(core_axis_name="c", subcore_axis_name="s")

    @functools.partial(
        pl.kernel,
        out_type=jax.ShapeDtypeStruct((HEADS_TOTAL, N_TOK, D_HEAD), jnp.float32),
        mesh=mesh,
        compiler_params=pltpu.CompilerParams(use_tc_tiling_on_sc=False),
        scratch_types=[
            pltpu.VMEM((CHUNK,), jnp.uint32),
            pltpu.VMEM((CHUNK,), jnp.uint32),
            pltpu.VMEM((ROWS_PER_CHUNK,), jnp.int32),
            pltpu.VMEM((ROWS_PER_CHUNK, D_HEAD), jnp.float32),
            pltpu.SemaphoreType.DMA,
        ],
    )
    def sc_kernel(ids_hbm, prev_hbm, table_hbm, out_hbm, ids_v, prev_v, idx_v, rows_v, sem):
        wid = lax.axis_index("s") * 2 + lax.axis_index("c")
        for c in range(N_CHUNKS):
            tb = wid * TOK_PER_W + c * CHUNK
            pltpu.sync_copy(ids_hbm.at[pl.ds(tb, CHUNK)], ids_v)
            pltpu.sync_copy(prev_hbm.at[pl.ds(tb, CHUNK)], prev_v)
            for g in range(CHUNK // 16):
                cur = ids_v[pl.ds(g * 16, 16)]
                prv = prev_v[pl.ds(g * 16, 16)]
                acc2 = cur * jnp.uint32(1000003) + prv
                for head in range(HEADS_TOTAL):
                    n, h = divmod(head, NUM_HEADS)
                    mult = jnp.uint32((2654435761 + 40503 * (h + 1)) & 0xFFFFFFFF)
                    acc = acc2 if n else cur
                    hv = (acc * mult) >> jnp.uint32(8)
                    fid = (hv % jnp.uint32(TABLE)).astype(jnp.int32) + head * TABLE
                    idx_v[pl.ds(head * CHUNK + g * 16, 16)] = fid
            copies = []
            for j in range(HEADS_TOTAL):
                copies.append(
                    pltpu.async_copy(
                        table_hbm.at[idx_v.at[pl.ds(j * CHUNK, CHUNK)]],
                        rows_v.at[pl.ds(j * CHUNK, CHUNK)],
                        sem,
                    )
                )
            for cp in copies:
                cp.wait()
            for j in range(HEADS_TOTAL):
                pltpu.sync_copy(
                    rows_v.at[pl.ds(j * CHUNK, CHUNK)],
                    out_hbm.at[j, pl.ds(tb, CHUNK)],
                )

    return sc_kernel(ids_u32, prev_u32, emb_table)


def _tc_body(e0, e1, e2, e3, e4, e5, e6, e7, h_ref, perm_ref, wkT_ref, bk_ref,
             wvT_ref, bv_ref, qw_ref, kw_ref, cnw_ref, conv4_ref, o_ref,
             hist_ref):
    t = pl.program_id(1)

    @pl.when(t == 0)
    def _():
        hist_ref[...] = jnp.zeros_like(hist_ref)

    e_refs = (e0, e1, e2, e3, e4, e5, e6, e7)
    perm = perm_ref[...]
    k = bk_ref[...]
    v = bv_ref[...]
    for head in range(HEADS_TOTAL):
        x = e_refs[head][...]
        cat = jnp.concatenate([x[:, :D_HEAD], x[:, D_HEAD:]], axis=0)
        e_h = jnp.dot(perm, cat, preferred_element_type=jnp.float32)
        k = k + jnp.dot(e_h, wkT_ref[head], preferred_element_type=jnp.float32)
        v = v + jnp.dot(e_h, wvT_ref[head], preferred_element_type=jnp.float32)
    h = h_ref[0]
    qn = h * lax.rsqrt(jnp.mean(h * h, -1, keepdims=True) + EPS) * qw_ref[...]
    kn = k * lax.rsqrt(jnp.mean(k * k, -1, keepdims=True) + EPS) * kw_ref[...]
    dot = jnp.sum(qn * kn, -1, keepdims=True) * (1.0 / math.sqrt(D))
    s = jnp.sqrt(jnp.maximum(jnp.abs(dot), 1e-6)) * jnp.sign(dot)
    gate = jax.nn.sigmoid(s)
    gv = gate * v
    gn = gv * lax.rsqrt(jnp.mean(gv * gv, -1, keepdims=True) + EPS) * cnw_ref[...]
    full = jnp.concatenate([hist_ref[8 - HIST:8], gn], axis=0)
    yc = (conv4_ref[0:1] * full[0:TB]
          + conv4_ref[1:2] * full[2:TB + 2]
          + conv4_ref[2:3] * full[4:TB + 4]
          + conv4_ref[3:4] * full[6:TB + 6])
    o_ref[0] = yc * jax.nn.sigmoid(yc) + gv
    hist_ref[8 - HIST:8] = gn[TB - HIST:TB]


def _tc_dense(e_lin, hidden, perm, wkT3, bk2, wvT3, bv2, qw2, kw2, cnw2, conv4):
    grid = (B, NT)
    lines_per_head = N_TOK * D_HEAD // 128
    lines_per_batch = T * D_HEAD // 128
    e_specs = [
        pl.BlockSpec(
            (LB, 128),
            functools.partial(
                lambda head, b, t: (
                    (head * lines_per_head + b * lines_per_batch) // LB + t, 0),
                head),
        )
        for head in range(HEADS_TOTAL)
    ]
    return pl.pallas_call(
        _tc_body,
        grid=grid,
        in_specs=e_specs + [
            pl.BlockSpec((1, TB, D), lambda b, t: (b, t, 0)),
            pl.BlockSpec((TB, TB), lambda b, t: (0, 0)),
            pl.BlockSpec((HEADS_TOTAL, D_HEAD, D), lambda b, t: (0, 0, 0)),
            pl.BlockSpec((1, D), lambda b, t: (0, 0)),
            pl.BlockSpec((HEADS_TOTAL, D_HEAD, D), lambda b, t: (0, 0, 0)),
            pl.BlockSpec((1, D), lambda b, t: (0, 0)),
            pl.BlockSpec((1, D), lambda b, t: (0, 0)),
            pl.BlockSpec((1, D), lambda b, t: (0, 0)),
            pl.BlockSpec((1, D), lambda b, t: (0, 0)),
            pl.BlockSpec((KSZ, D), lambda b, t: (0, 0)),
        ],
        out_specs=pl.BlockSpec((1, TB, D), lambda b, t: (b, t, 0)),
        out_shape=jax.ShapeDtypeStruct((B, T, D), jnp.float32),
        scratch_shapes=[pltpu.VMEM((8, D), jnp.float32)],
    )(*([e_lin] * HEADS_TOTAL), hidden, perm, wkT3, bk2, wvT3, bv2, qw2, kw2,
      cnw2, conv4)


def kernel(hidden_states, input_ids, emb_table, Wk, bk, Wv, bv, qn_w, kn_w, cn_w, conv_w):
    ids_u32 = input_ids.astype(jnp.uint32).reshape(-1)
    prev_u32 = jnp.concatenate(
        [jnp.zeros((B, 1), jnp.uint32), input_ids[:, :-1].astype(jnp.uint32)], axis=1
    ).reshape(-1)
    e_hm = _sc_hash_gather(ids_u32, prev_u32, emb_table)
    e_lin = e_hm.reshape(N_LINES, 128)
    tt = jnp.arange(TB)
    src = jnp.where(tt % 2 == 0, tt // 2, TB // 2 + tt // 2)
    perm = jax.nn.one_hot(src, TB, dtype=jnp.float32)
    return _tc_dense(
        e_lin,
        hidden_states,
        perm,
        Wk.T.reshape(HEADS_TOTAL, D_HEAD, D),
        bk.reshape(1, D),
        Wv.T.reshape(HEADS_TOTAL, D_HEAD, D),
        bv.reshape(1, D),
        qn_w.reshape(1, D),
        kn_w.reshape(1, D),
        cn_w.reshape(1, D),
        jnp.transpose(conv_w[:, 0, :]),
    )
